# baseline (speedup 1.0000x reference)
import functools

import jax
import jax.numpy as jnp
from jax.experimental import pallas as pl

_EDGE_LIST = [(0, 9), (0, 10), (0, 5), (9, 7), (9, 8), (9, 4), (2, 10), (2, 5),
              (7, 9), (7, 5), (7, 4), (8, 9), (8, 4), (6, 0), (6, 2), (6, 5),
              (6, 9), (3, 10), (3, 5), (10, 5), (1, 0), (1, 3)]
_N = 11
_SRC = tuple(e[0] for e in _EDGE_LIST) + tuple(range(_N))
_DST = tuple(e[1] for e in _EDGE_LIST) + tuple(range(_N))
_E = len(_SRC)
_IN_EDGES = tuple(tuple(k for k in range(_E) if _DST[k] == i) for i in range(_N))

_H, _C = 2, 32
_HC = _H * _C
_HID = 32


def _elu(v):
    return jnp.where(v > 0, v, jnp.exp(jnp.minimum(v, 0.0)) - 1.0)


def _mm(a, w):
    return jnp.dot(a, w, preferred_element_type=jnp.float32)


def _gat_packed(xp, Wl, bl, Wr, br, Ab, bias, bb2):
    xl = (_mm(xp, Wl) + bl).reshape(_N, bb2, 2 * _HC)
    xr = (_mm(xp, Wr) + br).reshape(_N, bb2, 2 * _HC)
    ex = []
    m = []
    for k in range(_E):
        e_k = jax.nn.leaky_relu(xl[_SRC[k]] + xr[_DST[k]], negative_slope=0.2)
        ex_k = jnp.exp(_mm(e_k, Ab))
        ex.append(ex_k)
        m.append(ex_k * xl[_SRC[k]])
    rows = []
    for i in range(_N):
        ks = _IN_EDGES[i]
        sm = ex[ks[0]]
        acc = m[ks[0]]
        for k in ks[1:]:
            sm = sm + ex[k]
            acc = acc + m[k]
        rows.append(acc / (sm + 1e-16))
    return jnp.stack(rows) + bias


def _body(x_ref, h0_ref, Wl1, bl1, Wr1, br1, Ab1, b1, Wl2, bl2, Wr2, br2,
          Ab2, b2, Wir, bir, Wiz, biz, Win, bin_, Whr, bhr, Whz, bhz,
          Whn, bhn, Hw1, Hb1, Hw2, Hb2, out_ref, hnew_ref):
    bb = x_ref.shape[0]
    bb2 = bb // 2
    d = x_ref.shape[2]
    xw = x_ref[...].reshape(bb, _N * d)
    xp = jnp.stack(
        [jnp.concatenate([xw[:bb2, n * d:(n + 1) * d],
                          xw[bb2:, n * d:(n + 1) * d]], axis=1)
         for n in range(_N)]).reshape(_N * bb2, 2 * d)

    h = _gat_packed(xp, Wl1[...], bl1[...], Wr1[...], br1[...], Ab1[...],
                    b1[...], bb2)
    h = _elu(h)
    h = _gat_packed(h.reshape(_N * bb2, 2 * _HC), Wl2[...], bl2[...],
                    Wr2[...], br2[...], Ab2[...], b2[...], bb2)
    h = _elu(h)

    gx = h.reshape(_N * bb2, 2 * _HC)
    h0w = h0_ref[...]
    h0p = jnp.stack(
        [jnp.concatenate([h0w[:bb2, n * _HID:(n + 1) * _HID],
                          h0w[bb2:, n * _HID:(n + 1) * _HID]], axis=1)
         for n in range(_N)]).reshape(_N * bb2, 2 * _HID)

    r = jax.nn.sigmoid(_mm(gx, Wir[...]) + bir[...] + _mm(h0p, Whr[...]) + bhr[...])
    z = jax.nn.sigmoid(_mm(gx, Wiz[...]) + biz[...] + _mm(h0p, Whz[...]) + bhz[...])
    n_ = jnp.tanh(_mm(gx, Win[...]) + bin_[...]
                  + r * (_mm(h0p, Whn[...]) + bhn[...]))
    hnew = (1.0 - z) * n_ + z * h0p
    t = hnew.reshape(_N, bb2, 2 * _HID)
    hnew_ref[...] = jnp.concatenate(
        [jnp.concatenate([t[i][:, :_HID], t[i][:, _HID:]], axis=0)
         for i in range(_N)], axis=1)

    outs = []
    for i in range(_N):
        h1 = jax.nn.relu(_mm(t[i], Hw1[i]) + Hb1[i])
        op = _mm(h1, Hw2[i]) + Hb2[i]
        outs.append(jnp.concatenate([op[:, :3], op[:, 3:]], axis=0))
    o = jnp.concatenate(outs, axis=1)
    ot = jnp.tanh(o)
    osig = jax.nn.sigmoid(o)
    lane = jax.lax.broadcasted_iota(jnp.int32, o.shape, 1) % 3
    out_ref[...] = jnp.where(lane == 2, osig,
                             jnp.where(lane == 0, ot * 0.3, ot * 0.2))


def _att_mat(att):
    z = jnp.zeros((_C, _C), jnp.float32)
    blocks = []
    for h in range(_H):
        row = [z] * _H
        row[h] = jnp.broadcast_to(att[h][:, None], (_C, _C))
        blocks.append(jnp.concatenate(row, axis=1))
    return jnp.concatenate(blocks, axis=0)


def _bd2(w):
    a, b = w.shape
    z = jnp.zeros((a, b), w.dtype)
    return jnp.concatenate([jnp.concatenate([w, z], axis=1),
                            jnp.concatenate([z, w], axis=1)], axis=0)


def kernel(x, hidden_state, edge_index, params):
    B, N, D = x.shape
    p = params
    bb = 512
    h0w = hidden_state.reshape(B, N * _HID)

    t2 = lambda v: jnp.concatenate([v, v]).reshape(1, -1)
    WihT, WhhT = p['Wih'].T, p['Whh'].T
    bih, bhh = p['bih'], p['bhh']
    Hw1p = jnp.zeros((N, 2 * _HID, 32), jnp.float32)
    Hw1p = Hw1p.at[:, :_HID, :16].set(p['Hw1']).at[:, _HID:, 16:].set(p['Hw1'])
    Hw2p = jnp.zeros((N, 32, 6), jnp.float32)
    Hw2p = Hw2p.at[:, :16, :3].set(p['Hw2']).at[:, 16:, 3:].set(p['Hw2'])
    Hb1p = jnp.concatenate([p['Hb1'], p['Hb1']], axis=1).reshape(N, 1, 32)
    Hb2p = jnp.concatenate([p['Hb2'], p['Hb2']], axis=1).reshape(N, 1, 6)

    weights = [
        _bd2(p['Wl1']), t2(p['bl1']), _bd2(p['Wr1']), t2(p['br1']),
        _bd2(_att_mat(p['att1'])), t2(p['bias1']),
        _bd2(p['Wl2']), t2(p['bl2']), _bd2(p['Wr2']), t2(p['br2']),
        _bd2(_att_mat(p['att2'])), t2(p['bias2']),
        _bd2(WihT[:, :32]), t2(bih[:32]), _bd2(WihT[:, 32:64]), t2(bih[32:64]),
        _bd2(WihT[:, 64:]), t2(bih[64:]),
        _bd2(WhhT[:, :32]), t2(bhh[:32]), _bd2(WhhT[:, 32:64]), t2(bhh[32:64]),
        _bd2(WhhT[:, 64:]), t2(bhh[64:]),
        Hw1p, Hb1p, Hw2p, Hb2p,
    ]

    grid = (B // bb,)
    w_specs = [pl.BlockSpec(w.shape, (lambda nd: (lambda i: (0,) * nd))(w.ndim))
               for w in weights]

    out3, hnew_w = pl.pallas_call(
        _body,
        grid=grid,
        in_specs=[pl.BlockSpec((bb, N, D), lambda i: (i, 0, 0)),
                  pl.BlockSpec((bb, N * _HID), lambda i: (i, 0))] + w_specs,
        out_specs=[pl.BlockSpec((bb, N * 3), lambda i: (i, 0)),
                   pl.BlockSpec((bb, N * _HID), lambda i: (i, 0))],
        out_shape=[jax.ShapeDtypeStruct((B, N * 3), jnp.float32),
                   jax.ShapeDtypeStruct((B, N * _HID), jnp.float32)],
    )(x, h0w, *weights)

    return out3.reshape(B, N, 3), hnew_w.reshape(1, B * N, _HID)


# speedup vs baseline: 1.1284x; 1.1284x over previous
import functools

import jax
import jax.numpy as jnp
from jax.experimental import pallas as pl

_EDGE_LIST = [(0, 9), (0, 10), (0, 5), (9, 7), (9, 8), (9, 4), (2, 10), (2, 5),
              (7, 9), (7, 5), (7, 4), (8, 9), (8, 4), (6, 0), (6, 2), (6, 5),
              (6, 9), (3, 10), (3, 5), (10, 5), (1, 0), (1, 3)]
_N = 11
_SRC = tuple(e[0] for e in _EDGE_LIST) + tuple(range(_N))
_DST = tuple(e[1] for e in _EDGE_LIST) + tuple(range(_N))
_E = len(_SRC)
_IN_EDGES = tuple(tuple(k for k in range(_E) if _DST[k] == i) for i in range(_N))

_H, _C = 2, 32
_HC = _H * _C
_HID = 32


def _elu(v):
    return jnp.maximum(v, jnp.exp(jnp.minimum(v, 0.0)) - 1.0)


def _mm(a, w):
    return jnp.dot(a, w, preferred_element_type=jnp.float32)


def _gat_packed(xp, Wl, bl, Wr, br, Ab, bias, bb2):
    xl = (_mm(xp, Wl) + bl).reshape(_N, bb2, 2 * _HC)
    xr = (_mm(xp, Wr) + br).reshape(_N, bb2, 2 * _HC)
    ex = []
    m = []
    for k in range(_E):
        s_k = xl[_SRC[k]] + xr[_DST[k]]
        e_k = jnp.maximum(s_k, 0.2 * s_k)
        ex_k = jnp.exp(_mm(e_k, Ab))
        ex.append(ex_k)
        m.append(ex_k * xl[_SRC[k]])
    rows = []
    for i in range(_N):
        ks = _IN_EDGES[i]
        sm = ex[ks[0]]
        acc = m[ks[0]]
        for k in ks[1:]:
            sm = sm + ex[k]
            acc = acc + m[k]
        rows.append(acc / (sm + 1e-16))
    return jnp.stack(rows) + bias


def _body(x_ref, h0_ref, Wl1, bl1, Wr1, br1, Ab1, b1, Wl2, bl2, Wr2, br2,
          Ab2, b2, Wir, bir, Wiz, biz, Win, bin_, Whr, bhr, Whz, bhz,
          Whn, bhn, Hw1, Hb1, Hw2, Hb2, out_ref, hnew_ref):
    bb = x_ref.shape[0]
    bb2 = bb // 2
    d = x_ref.shape[1] // _N
    xw = x_ref[...]
    xp = jnp.stack(
        [jnp.concatenate([xw[:bb2, n * d:(n + 1) * d],
                          xw[bb2:, n * d:(n + 1) * d]], axis=1)
         for n in range(_N)]).reshape(_N * bb2, 2 * d)

    h = _gat_packed(xp, Wl1[...], bl1[...], Wr1[...], br1[...], Ab1[...],
                    b1[...], bb2)
    h = _elu(h)
    h = _gat_packed(h.reshape(_N * bb2, 2 * _HC), Wl2[...], bl2[...],
                    Wr2[...], br2[...], Ab2[...], b2[...], bb2)
    h = _elu(h)

    gx = h.reshape(_N * bb2, 2 * _HC)
    h0w = h0_ref[...]
    h0p = jnp.stack(
        [jnp.concatenate([h0w[:bb2, n * _HID:(n + 1) * _HID],
                          h0w[bb2:, n * _HID:(n + 1) * _HID]], axis=1)
         for n in range(_N)]).reshape(_N * bb2, 2 * _HID)

    r = jax.nn.sigmoid(_mm(gx, Wir[...]) + (_mm(h0p, Whr[...]) + bhr[...]))
    z = jax.nn.sigmoid(_mm(gx, Wiz[...]) + (_mm(h0p, Whz[...]) + bhz[...]))
    n_ = jnp.tanh(_mm(gx, Win[...]) + bin_[...]
                  + r * (_mm(h0p, Whn[...]) + bhn[...]))
    hnew = (1.0 - z) * n_ + z * h0p
    t = hnew.reshape(_N, bb2, 2 * _HID)
    hnew_ref[...] = jnp.concatenate(
        [jnp.concatenate([t[i][:, :_HID], t[i][:, _HID:]], axis=0)
         for i in range(_N)], axis=1)

    outs = []
    for i in range(_N):
        h1 = jax.nn.relu(_mm(t[i], Hw1[i]) + Hb1[i])
        op = _mm(h1, Hw2[i]) + Hb2[i]
        outs.append(jnp.concatenate([op[:, :3], op[:, 3:]], axis=0))
    o = jnp.concatenate(outs, axis=1)
    ot = jnp.tanh(o)
    osig = jax.nn.sigmoid(o)
    lane = jax.lax.broadcasted_iota(jnp.int32, o.shape, 1) % 3
    out_ref[...] = jnp.where(lane == 2, osig,
                             jnp.where(lane == 0, ot * 0.3, ot * 0.2))


def _att_mat(att):
    z = jnp.zeros((_C, _C), jnp.float32)
    blocks = []
    for h in range(_H):
        row = [z] * _H
        row[h] = jnp.broadcast_to(att[h][:, None], (_C, _C))
        blocks.append(jnp.concatenate(row, axis=1))
    return jnp.concatenate(blocks, axis=0)


def _bd2(w):
    a, b = w.shape
    z = jnp.zeros((a, b), w.dtype)
    return jnp.concatenate([jnp.concatenate([w, z], axis=1),
                            jnp.concatenate([z, w], axis=1)], axis=0)


def kernel(x, hidden_state, edge_index, params):
    B, N, D = x.shape
    p = params
    bb = 512
    xw = x.reshape(B, N * D)
    h0w = hidden_state.reshape(B, N * _HID)

    t2 = lambda v: jnp.concatenate([v, v]).reshape(1, -1)
    WihT, WhhT = p['Wih'].T, p['Whh'].T
    bih, bhh = p['bih'], p['bhh']
    Hw1p = jnp.zeros((N, 2 * _HID, 32), jnp.float32)
    Hw1p = Hw1p.at[:, :_HID, :16].set(p['Hw1']).at[:, _HID:, 16:].set(p['Hw1'])
    Hw2p = jnp.zeros((N, 32, 6), jnp.float32)
    Hw2p = Hw2p.at[:, :16, :3].set(p['Hw2']).at[:, 16:, 3:].set(p['Hw2'])
    Hb1p = jnp.concatenate([p['Hb1'], p['Hb1']], axis=1).reshape(N, 1, 32)
    Hb2p = jnp.concatenate([p['Hb2'], p['Hb2']], axis=1).reshape(N, 1, 6)

    weights = [
        _bd2(p['Wl1']), t2(p['bl1']), _bd2(p['Wr1']), t2(p['br1']),
        _bd2(_att_mat(p['att1'])), t2(p['bias1']),
        _bd2(p['Wl2']), t2(p['bl2']), _bd2(p['Wr2']), t2(p['br2']),
        _bd2(_att_mat(p['att2'])), t2(p['bias2']),
        _bd2(WihT[:, :32]), t2(bih[:32]), _bd2(WihT[:, 32:64]), t2(bih[32:64]),
        _bd2(WihT[:, 64:]), t2(bih[64:]),
        _bd2(WhhT[:, :32]), t2(bhh[:32] + bih[:32]), _bd2(WhhT[:, 32:64]), t2(bhh[32:64] + bih[32:64]),
        _bd2(WhhT[:, 64:]), t2(bhh[64:]),
        Hw1p, Hb1p, Hw2p, Hb2p,
    ]

    grid = (B // bb,)
    batch_spec = lambda f: pl.BlockSpec((bb, f), lambda i: (i, 0))
    w_specs = [pl.BlockSpec(w.shape, (lambda nd: (lambda i: (0,) * nd))(w.ndim))
               for w in weights]

    out_w, hnew_w = pl.pallas_call(
        _body,
        grid=grid,
        in_specs=[batch_spec(N * D), batch_spec(N * _HID)] + w_specs,
        out_specs=[batch_spec(N * 3), batch_spec(N * _HID)],
        out_shape=[jax.ShapeDtypeStruct((B, N * 3), jnp.float32),
                   jax.ShapeDtypeStruct((B, N * _HID), jnp.float32)],
    )(xw, h0w, *weights)

    out = out_w.reshape(B, N, 3)
    hnew = hnew_w.reshape(1, B * N, _HID)
    return out, hnew
